# Initial kernel scaffold; baseline (speedup 1.0000x reference)
import jax
import jax.numpy as jnp
from jax.experimental import pallas as pl


def kernel(x_0, node_idx, edge_idx, W0_l0, b01_l0, W1_l0, b10_l0, W0_l1, b01_l1, W1_l1, b10_l1, W_lin, b_lin):
    raise NotImplementedError("write your pallas kernel here")


# speedup vs baseline: 5.0328x; 5.0328x over previous
import functools

import jax
import jax.numpy as jnp
from jax import lax
from jax.experimental import pallas as pl
from jax.experimental.pallas import tpu as pltpu
from jax.experimental.pallas import tpu_sc as plsc

NC, NS, L = 2, 16, 16
N_NNZ = 160000
NE = 10000
P = 10240
SEG = 640
CH = 80
PER_SUB = N_NNZ // NS
NCHUNK = PER_SUB // CH
BLK = 1024
HID = 256
HHID = 128

_mesh = plsc.VectorSubcoreMesh(
    core_axis_name="c", subcore_axis_name="s", num_cores=NC, num_subcores=NS
)


def _rsqrt16(x):
    i = lax.bitcast_convert_type(x, jnp.int32)
    i = jnp.int32(0x5F3759DF) - lax.shift_right_logical(i, 1)
    y = lax.bitcast_convert_type(i, jnp.float32)
    for _ in range(3):
        y = y * (1.5 - 0.5 * x * y * y)
    return y


def _fill1d(ref, n, val):
    def body(j, _):
        ref[pl.ds(j * L, L)] = jnp.full((L,), val, jnp.float32)
        return 0
    lax.fori_loop(0, n // L, body, 0)


def _cards_body(icat, cards_out, acc, idxb, onesb, workb):
    c = lax.axis_index("c")
    s = lax.axis_index("s")
    _fill1d(workb, SEG, 0.0)
    pltpu.sync_copy(workb, acc.at[pl.ds(s * SEG, SEG)])
    _fill1d(onesb, CH, 1.0)
    plsc.subcore_barrier()
    ibase = c * N_NNZ + s * PER_SUB

    def chunk(k, _):
        pltpu.sync_copy(icat.at[pl.ds(ibase + k * CH, CH)], idxb)
        pltpu.sync_copy(onesb, acc.at[idxb], add=True)
        return 0

    lax.fori_loop(0, NCHUNK, chunk, 0)
    plsc.subcore_barrier()
    pltpu.sync_copy(acc.at[pl.ds(s * SEG, SEG)], workb)
    m = c.astype(jnp.float32)

    def post(j, _):
        x = workb[pl.ds(j * L, L)]
        y = _rsqrt16(x)
        card = y * ((1.0 - m) + m * y * y)
        workb[pl.ds(j * L, L)] = jnp.where(x > 0, card, 0.0)
        return 0

    lax.fori_loop(0, SEG // L, post, 0)
    pltpu.sync_copy(workb, cards_out.at[pl.ds(c * P + s * SEG, SEG)])


_cards_call = pl.kernel(
    _cards_body,
    out_type=jax.ShapeDtypeStruct((2 * P,), jnp.float32),
    mesh=_mesh,
    scratch_types=[
        pltpu.VMEM_SHARED((P,), jnp.float32),
        pltpu.VMEM((CH,), jnp.int32),
        pltpu.VMEM((CH,), jnp.float32),
        pltpu.VMEM((SEG,), jnp.float32),
    ],
)


def _inv_body(icat, cards, invs_out, acc, gidx, didx, valb, workb, sem):
    c = lax.axis_index("c")
    s = lax.axis_index("s")
    _fill1d(workb, SEG, 0.0)
    pltpu.sync_copy(workb, acc.at[pl.ds(s * SEG, SEG)])
    plsc.subcore_barrier()
    gbase = (1 - c) * N_NNZ + s * PER_SUB
    dbase = c * N_NNZ + s * PER_SUB
    goff = (1 - c) * P

    def chunk(k, _):
        pltpu.sync_copy(icat.at[pl.ds(gbase + k * CH, CH)], gidx)
        for j in range(CH // L):
            gidx[pl.ds(j * L, L)] = gidx[pl.ds(j * L, L)] + goff
        pltpu.sync_copy(icat.at[pl.ds(dbase + k * CH, CH)], didx)
        pltpu.async_copy(cards.at[gidx], valb, sem).wait()
        pltpu.sync_copy(valb, acc.at[didx], add=True)
        return 0

    lax.fori_loop(0, NCHUNK, chunk, 0)
    plsc.subcore_barrier()
    pltpu.sync_copy(acc.at[pl.ds(s * SEG, SEG)], workb)

    def post(j, _):
        x = workb[pl.ds(j * L, L)]
        y = _rsqrt16(x)
        workb[pl.ds(j * L, L)] = jnp.where(x > 0, y * y, 0.0)
        return 0

    lax.fori_loop(0, SEG // L, post, 0)
    pltpu.sync_copy(workb, invs_out.at[pl.ds(c * P + s * SEG, SEG)])


_inv_call = pl.kernel(
    _inv_body,
    out_type=jax.ShapeDtypeStruct((2 * P,), jnp.float32),
    mesh=_mesh,
    scratch_types=[
        pltpu.VMEM_SHARED((P,), jnp.float32),
        pltpu.VMEM((CH,), jnp.int32),
        pltpu.VMEM((CH,), jnp.int32),
        pltpu.VMEM((CH,), jnp.float32),
        pltpu.VMEM((SEG,), jnp.float32),
        pltpu.SemaphoreType.DMA,
    ],
)


def _spass_body(gbase, dbase, src, icat, out, acc, sidx, didx, rows, sem):
    c = lax.axis_index("c")
    s = lax.axis_index("s")

    def zrow(r, _):
        for j in range(HHID // L):
            rows[r, pl.ds(j * L, L)] = jnp.zeros((L,), jnp.float32)
        return 0

    lax.fori_loop(0, CH, zrow, 0)
    for t in range(SEG // CH):
        pltpu.sync_copy(rows, acc.at[pl.ds(s * SEG + t * CH, CH)])
    plsc.subcore_barrier()
    coff = c * P

    def chunk(k, _):
        base = s * PER_SUB + k * CH
        pltpu.sync_copy(icat.at[pl.ds(gbase + base, CH)], sidx)
        for j in range(CH // L):
            sidx[pl.ds(j * L, L)] = sidx[pl.ds(j * L, L)] + coff
        pltpu.sync_copy(icat.at[pl.ds(dbase + base, CH)], didx)
        pltpu.async_copy(src.at[sidx], rows, sem).wait()
        pltpu.sync_copy(rows, acc.at[didx], add=True)
        return 0

    lax.fori_loop(0, NCHUNK, chunk, 0)
    plsc.subcore_barrier()
    pltpu.sync_copy(
        acc.at[pl.ds(s * SEG, SEG)], out.at[pl.ds(c * P + s * SEG, SEG)]
    )


def _make_spass(gbase, dbase):
    return pl.kernel(
        functools.partial(_spass_body, gbase, dbase),
        out_type=jax.ShapeDtypeStruct((2 * P, HHID), jnp.float32),
        mesh=_mesh,
        scratch_types=[
            pltpu.VMEM_SHARED((P, HHID), jnp.float32),
            pltpu.VMEM((CH,), jnp.int32),
            pltpu.VMEM((CH,), jnp.int32),
            pltpu.VMEM((CH, HHID), jnp.float32),
            pltpu.SemaphoreType.DMA,
        ],
    )


_spass_n2e = _make_spass(0, N_NNZ)
_spass_e2n = _make_spass(N_NNZ, 0)


def _mm_split_body(x_ref, sc_ref, w_ref, o_ref):
    res = jnp.dot(x_ref[...], w_ref[...], preferred_element_type=jnp.float32)
    res = res * sc_ref[...]
    o_ref[0] = res[:, :HHID]
    o_ref[1] = res[:, HHID:]


_mm_call = pl.pallas_call(
    _mm_split_body,
    grid=(P // BLK,),
    in_specs=[
        pl.BlockSpec((BLK, HID), lambda i: (i, 0)),
        pl.BlockSpec((BLK, 1), lambda i: (i, 0)),
        pl.BlockSpec((HID, HID), lambda i: (0, 0)),
    ],
    out_specs=pl.BlockSpec((2, BLK, HHID), lambda i: (0, i, 0)),
    out_shape=jax.ShapeDtypeStruct((2, P, HHID), jnp.float32),
)


def _mid_body(raw_ref, d_ref, b_ref, w_ref, p_ref, o_ref):
    lo = jnp.maximum(raw_ref[0] * d_ref[...] + b_ref[:, :HHID], 0.0)
    hi = jnp.maximum(raw_ref[1] * d_ref[...] + b_ref[:, HHID:], 0.0)
    x = jnp.concatenate([lo, hi], axis=1)
    z = jnp.dot(x, w_ref[...], preferred_element_type=jnp.float32)
    z = z * p_ref[...]
    o_ref[0] = z[:, :HHID]
    o_ref[1] = z[:, HHID:]


_mid_call = pl.pallas_call(
    _mid_body,
    grid=(P // BLK,),
    in_specs=[
        pl.BlockSpec((2, BLK, HHID), lambda i: (0, i, 0)),
        pl.BlockSpec((BLK, 1), lambda i: (i, 0)),
        pl.BlockSpec((1, HID), lambda i: (0, 0)),
        pl.BlockSpec((HID, HID), lambda i: (0, 0)),
        pl.BlockSpec((BLK, 1), lambda i: (i, 0)),
    ],
    out_specs=pl.BlockSpec((2, BLK, HHID), lambda i: (0, i, 0)),
    out_shape=jax.ShapeDtypeStruct((2, P, HHID), jnp.float32),
)


def _final_body(raw_ref, d_ref, b_ref, wl_ref, bl_ref, o_ref, mx_ref):
    i = pl.program_id(0)
    lo = jnp.maximum(raw_ref[0] * d_ref[...] + b_ref[:, :HHID], 0.0)
    hi = jnp.maximum(raw_ref[1] * d_ref[...] + b_ref[:, HHID:], 0.0)
    x = jnp.concatenate([lo, hi], axis=1)
    rowid = i * BLK + lax.broadcasted_iota(jnp.int32, (BLK, 1), 0)
    x = jnp.where(rowid < NE, x, -jnp.inf)
    bm = jnp.max(x, axis=0, keepdims=True)

    @pl.when(i == 0)
    def _():
        mx_ref[0:1] = bm

    @pl.when(i > 0)
    def _():
        mx_ref[0:1] = jnp.maximum(mx_ref[0:1], bm)

    @pl.when(i == P // BLK - 1)
    def _():
        prod = mx_ref[0:1] * wl_ref[...].reshape(1, HID)
        o_ref[...] = jnp.sum(prod, axis=1, keepdims=True) + bl_ref[...]


_final_call = pl.pallas_call(
    _final_body,
    grid=(P // BLK,),
    in_specs=[
        pl.BlockSpec((2, BLK, HHID), lambda i: (0, i, 0)),
        pl.BlockSpec((BLK, 1), lambda i: (i, 0)),
        pl.BlockSpec((1, HID), lambda i: (0, 0)),
        pl.BlockSpec((HID, 1), lambda i: (0, 0)),
        pl.BlockSpec((1, 1), lambda i: (0, 0)),
    ],
    out_specs=pl.BlockSpec((1, 1), lambda i: (0, 0)),
    out_shape=jax.ShapeDtypeStruct((1, 1), jnp.float32),
    scratch_shapes=[pltpu.VMEM((8, HID), jnp.float32)],
)


def kernel(x_0, node_idx, edge_idx, W0_l0, b01_l0, W1_l0, b10_l0,
           W0_l1, b01_l1, W1_l1, b10_l1, W_lin, b_lin):
    icat = jnp.concatenate(
        [node_idx.astype(jnp.int32), edge_idx.astype(jnp.int32)]
    )
    cards = _cards_call(icat)
    invs = _inv_call(icat, cards)
    node_card = cards[:P].reshape(P, 1)
    edge_card = cards[P:].reshape(P, 1)
    d0li = invs[:P].reshape(P, 1)
    d1li = invs[P:].reshape(P, 1)

    x0p = jnp.pad(x_0, ((0, P - NE), (0, 0)))
    b01_0 = b01_l0.reshape(1, HID)
    b10_0 = b10_l0.reshape(1, HID)
    b01_1 = b01_l1.reshape(1, HID)
    b10_1 = b10_l1.reshape(1, HID)

    xm = _mm_call(x0p, node_card, W0_l0)
    r1 = _spass_n2e(xm.reshape(2 * P, HHID), icat)
    ym = _mid_call(r1.reshape(2, P, HHID), d1li, b01_0, W1_l0, edge_card)
    r2 = _spass_e2n(ym.reshape(2 * P, HHID), icat)
    xm2 = _mid_call(r2.reshape(2, P, HHID), d0li, b10_0, W0_l1, node_card)
    r3 = _spass_n2e(xm2.reshape(2 * P, HHID), icat)
    ym2 = _mid_call(r3.reshape(2, P, HHID), d1li, b01_1, W1_l1, edge_card)
    r4 = _spass_e2n(ym2.reshape(2 * P, HHID), icat)
    out = _final_call(
        r4.reshape(2, P, HHID), d0li, b10_1, W_lin, b_lin.reshape(1, 1)
    )
    return out.reshape(1)


def kernel(x_0, node_idx, edge_idx, W0_l0, b01_l0, W1_l0, b10_l0,
           W0_l1, b01_l1, W1_l1, b10_l1, W_lin, b_lin):
    icat = jnp.concatenate(
        [node_idx.astype(jnp.int32), edge_idx.astype(jnp.int32)]
    )
    cards = _cards_call(icat)
    invs = _inv_call(icat, cards)
    node_card = cards[:P].reshape(P, 1)
    edge_card = cards[P:].reshape(P, 1)
    d0li = invs[:P].reshape(P, 1)
    d1li = invs[P:].reshape(P, 1)

    x0p = jnp.pad(x_0, ((0, P - NE), (0, 0)))
    b01_0 = b01_l0.reshape(1, HID)
    b10_0 = b10_l0.reshape(1, HID)
    b01_1 = b01_l1.reshape(1, HID)
    b10_1 = b10_l1.reshape(1, HID)

    xm = _mm_call(x0p, node_card, W0_l0)
    r1 = _spass_n2e(xm.reshape(2 * P, HHID), icat)
    ym = _mid_call(r1.reshape(2, P, HHID), d1li, b01_0, W1_l0, edge_card)
    r2 = _spass_e2n(ym.reshape(2 * P, HHID), icat)
    xm2 = _mid_call(r2.reshape(2, P, HHID), d0li, b10_0, W0_l1, node_card)
    r3 = _spass_n2e(xm2.reshape(2 * P, HHID), icat)
    ym2 = _mid_call(r3.reshape(2, P, HHID), d1li, b01_1, W1_l1, edge_card)
    r4 = _spass_e2n(ym2.reshape(2 * P, HHID), icat)
    out = _final_call(
        r4.reshape(2, P, HHID), d0li, b10_1, W_lin, b_lin.reshape(1, 1)
    )
    return out.reshape(1)


# speedup vs baseline: 12.2111x; 2.4263x over previous
import functools

import jax
import jax.numpy as jnp
from jax import lax
from jax.experimental import pallas as pl
from jax.experimental.pallas import tpu as pltpu
from jax.experimental.pallas import tpu_sc as plsc

NC, NS, L = 2, 16, 16
N_NNZ = 160000
NE = 10000
P = 10240
SEG = 640
CH = 80
PER_SUB = N_NNZ // NS
NCHUNK = PER_SUB // CH
BLK = 1024
HID = 256
HHID = 128

_mesh = plsc.VectorSubcoreMesh(
    core_axis_name="c", subcore_axis_name="s", num_cores=NC, num_subcores=NS
)


def _rsqrt16(x):
    i = lax.bitcast_convert_type(x, jnp.int32)
    i = jnp.int32(0x5F3759DF) - lax.shift_right_logical(i, 1)
    y = lax.bitcast_convert_type(i, jnp.float32)
    for _ in range(3):
        y = y * (1.5 - 0.5 * x * y * y)
    return y


def _fill1d(ref, n, val):
    def body(j, _):
        ref[pl.ds(j * L, L)] = jnp.full((L,), val, jnp.float32)
        return 0
    lax.fori_loop(0, n // L, body, 0)


def _cards_body(icat, cards_out, acc, didx0, didx1, onesb, workb,
                semd0, semd1):
    c = lax.axis_index("c")
    s = lax.axis_index("s")
    _fill1d(workb, SEG, 0.0)
    pltpu.sync_copy(workb, acc.at[pl.ds(s * SEG, SEG)])
    _fill1d(onesb, CH, 1.0)
    ibase = c * N_NNZ + s * PER_SUB
    plsc.subcore_barrier()
    pltpu.async_copy(icat.at[pl.ds(ibase, CH)], didx0, semd0)

    def chunk(k, _):
        nxt = k + 1

        @pl.when(jnp.logical_and(nxt < NCHUNK, nxt % 2 == 1))
        def _():
            pltpu.async_copy(icat.at[pl.ds(ibase + nxt * CH, CH)], didx1, semd1)

        @pl.when(jnp.logical_and(nxt < NCHUNK, nxt % 2 == 0))
        def _():
            pltpu.async_copy(icat.at[pl.ds(ibase + nxt * CH, CH)], didx0, semd0)

        @pl.when(k % 2 == 0)
        def _():
            pltpu.make_async_copy(icat.at[pl.ds(ibase, CH)], didx0, semd0).wait()
            pltpu.sync_copy(onesb, acc.at[didx0], add=True)

        @pl.when(k % 2 == 1)
        def _():
            pltpu.make_async_copy(icat.at[pl.ds(ibase, CH)], didx1, semd1).wait()
            pltpu.sync_copy(onesb, acc.at[didx1], add=True)

        return 0

    lax.fori_loop(0, NCHUNK, chunk, 0)
    plsc.subcore_barrier()
    pltpu.sync_copy(acc.at[pl.ds(s * SEG, SEG)], workb)
    m = c.astype(jnp.float32)

    def post(j, _):
        x = workb[pl.ds(j * L, L)]
        y = _rsqrt16(x)
        card = y * ((1.0 - m) + m * y * y)
        workb[pl.ds(j * L, L)] = jnp.where(x > 0, card, 0.0)
        return 0

    lax.fori_loop(0, SEG // L, post, 0)
    pltpu.sync_copy(workb, cards_out.at[pl.ds(c * P + s * SEG, SEG)])


_cards_call = pl.kernel(
    _cards_body,
    out_type=jax.ShapeDtypeStruct((2 * P,), jnp.float32),
    mesh=_mesh,
    scratch_types=[
        pltpu.VMEM_SHARED((P,), jnp.float32),
        pltpu.VMEM((CH,), jnp.int32),
        pltpu.VMEM((CH,), jnp.int32),
        pltpu.VMEM((CH,), jnp.float32),
        pltpu.VMEM((SEG,), jnp.float32),
        pltpu.SemaphoreType.DMA,
        pltpu.SemaphoreType.DMA,
    ],
)


def _inv_body(icat, cards, invs_out, acc, gidx, didx0, didx1, val0, val1,
              workb, semg0, semg1, semd0, semd1):
    c = lax.axis_index("c")
    s = lax.axis_index("s")
    _fill1d(workb, SEG, 0.0)
    pltpu.sync_copy(workb, acc.at[pl.ds(s * SEG, SEG)])
    gbase = (1 - c) * N_NNZ + s * PER_SUB
    dbase = c * N_NNZ + s * PER_SUB
    pltpu.sync_copy(icat.at[pl.ds(gbase, PER_SUB)], gidx)
    goff = (1 - c) * P

    def addoff(r, _):
        gidx[pl.ds(r * L, L)] = gidx[pl.ds(r * L, L)] + goff
        return 0

    lax.fori_loop(0, PER_SUB // L, addoff, 0)
    plsc.subcore_barrier()
    pltpu.async_copy(cards.at[gidx.at[pl.ds(0, CH)]], val0, semg0)
    pltpu.async_copy(icat.at[pl.ds(dbase, CH)], didx0, semd0)

    def chunk(k, _):
        nxt = k + 1

        @pl.when(jnp.logical_and(nxt < NCHUNK, nxt % 2 == 1))
        def _():
            pltpu.async_copy(cards.at[gidx.at[pl.ds(nxt * CH, CH)]], val1, semg1)
            pltpu.async_copy(icat.at[pl.ds(dbase + nxt * CH, CH)], didx1, semd1)

        @pl.when(jnp.logical_and(nxt < NCHUNK, nxt % 2 == 0))
        def _():
            pltpu.async_copy(cards.at[gidx.at[pl.ds(nxt * CH, CH)]], val0, semg0)
            pltpu.async_copy(icat.at[pl.ds(dbase + nxt * CH, CH)], didx0, semd0)

        @pl.when(k % 2 == 0)
        def _():
            pltpu.make_async_copy(cards.at[gidx.at[pl.ds(0, CH)]], val0, semg0).wait()
            pltpu.make_async_copy(icat.at[pl.ds(dbase, CH)], didx0, semd0).wait()
            pltpu.sync_copy(val0, acc.at[didx0], add=True)

        @pl.when(k % 2 == 1)
        def _():
            pltpu.make_async_copy(cards.at[gidx.at[pl.ds(0, CH)]], val1, semg1).wait()
            pltpu.make_async_copy(icat.at[pl.ds(dbase, CH)], didx1, semd1).wait()
            pltpu.sync_copy(val1, acc.at[didx1], add=True)

        return 0

    lax.fori_loop(0, NCHUNK, chunk, 0)
    plsc.subcore_barrier()
    pltpu.sync_copy(acc.at[pl.ds(s * SEG, SEG)], workb)

    def post(j, _):
        x = workb[pl.ds(j * L, L)]
        y = _rsqrt16(x)
        workb[pl.ds(j * L, L)] = jnp.where(x > 0, y * y, 0.0)
        return 0

    lax.fori_loop(0, SEG // L, post, 0)
    pltpu.sync_copy(workb, invs_out.at[pl.ds(c * P + s * SEG, SEG)])


_inv_call = pl.kernel(
    _inv_body,
    out_type=jax.ShapeDtypeStruct((2 * P,), jnp.float32),
    mesh=_mesh,
    scratch_types=[
        pltpu.VMEM_SHARED((P,), jnp.float32),
        pltpu.VMEM((PER_SUB,), jnp.int32),
        pltpu.VMEM((CH,), jnp.int32),
        pltpu.VMEM((CH,), jnp.int32),
        pltpu.VMEM((CH,), jnp.float32),
        pltpu.VMEM((CH,), jnp.float32),
        pltpu.VMEM((SEG,), jnp.float32),
        pltpu.SemaphoreType.DMA,
        pltpu.SemaphoreType.DMA,
        pltpu.SemaphoreType.DMA,
        pltpu.SemaphoreType.DMA,
    ],
)


def _spass_body(gbase, dbase, src, icat, out, acc, sidx, didx0, didx1,
                rows0, rows1, semg0, semg1, semd0, semd1):
    c = lax.axis_index("c")
    s = lax.axis_index("s")

    def zrow(r, _):
        for j in range(HHID // L):
            rows0[r, pl.ds(j * L, L)] = jnp.zeros((L,), jnp.float32)
        return 0

    lax.fori_loop(0, CH, zrow, 0)
    for t in range(SEG // CH):
        pltpu.sync_copy(rows0, acc.at[pl.ds(s * SEG + t * CH, CH)])
    gnnz = gbase + s * PER_SUB
    dnnz = dbase + s * PER_SUB
    pltpu.sync_copy(icat.at[pl.ds(gnnz, PER_SUB)], sidx)
    coff = c * P

    def addoff(r, _):
        sidx[pl.ds(r * L, L)] = sidx[pl.ds(r * L, L)] + coff
        return 0

    lax.fori_loop(0, PER_SUB // L, addoff, 0)
    plsc.subcore_barrier()
    pltpu.async_copy(src.at[sidx.at[pl.ds(0, CH)]], rows0, semg0)
    pltpu.async_copy(icat.at[pl.ds(dnnz, CH)], didx0, semd0)

    def chunk(k, _):
        nxt = k + 1

        @pl.when(jnp.logical_and(nxt < NCHUNK, nxt % 2 == 1))
        def _():
            pltpu.async_copy(src.at[sidx.at[pl.ds(nxt * CH, CH)]], rows1, semg1)
            pltpu.async_copy(icat.at[pl.ds(dnnz + nxt * CH, CH)], didx1, semd1)

        @pl.when(jnp.logical_and(nxt < NCHUNK, nxt % 2 == 0))
        def _():
            pltpu.async_copy(src.at[sidx.at[pl.ds(nxt * CH, CH)]], rows0, semg0)
            pltpu.async_copy(icat.at[pl.ds(dnnz + nxt * CH, CH)], didx0, semd0)

        @pl.when(k % 2 == 0)
        def _():
            pltpu.make_async_copy(src.at[sidx.at[pl.ds(0, CH)]], rows0, semg0).wait()
            pltpu.make_async_copy(icat.at[pl.ds(dnnz, CH)], didx0, semd0).wait()
            pltpu.sync_copy(rows0, acc.at[didx0], add=True)

        @pl.when(k % 2 == 1)
        def _():
            pltpu.make_async_copy(src.at[sidx.at[pl.ds(0, CH)]], rows1, semg1).wait()
            pltpu.make_async_copy(icat.at[pl.ds(dnnz, CH)], didx1, semd1).wait()
            pltpu.sync_copy(rows1, acc.at[didx1], add=True)

        return 0

    lax.fori_loop(0, NCHUNK, chunk, 0)
    plsc.subcore_barrier()
    pltpu.sync_copy(
        acc.at[pl.ds(s * SEG, SEG)], out.at[pl.ds(c * P + s * SEG, SEG)]
    )


def _make_spass(gbase, dbase):
    return pl.kernel(
        functools.partial(_spass_body, gbase, dbase),
        out_type=jax.ShapeDtypeStruct((2 * P, HHID), jnp.float32),
        mesh=_mesh,
        scratch_types=[
            pltpu.VMEM_SHARED((P, HHID), jnp.float32),
            pltpu.VMEM((PER_SUB,), jnp.int32),
            pltpu.VMEM((CH,), jnp.int32),
            pltpu.VMEM((CH,), jnp.int32),
            pltpu.VMEM((CH, HHID), jnp.float32),
            pltpu.VMEM((CH, HHID), jnp.float32),
            pltpu.SemaphoreType.DMA,
            pltpu.SemaphoreType.DMA,
            pltpu.SemaphoreType.DMA,
            pltpu.SemaphoreType.DMA,
        ],
    )


_spass_n2e = _make_spass(0, N_NNZ)
_spass_e2n = _make_spass(N_NNZ, 0)


def _mm_split_body(x_ref, sc_ref, w_ref, o_ref):
    res = jnp.dot(x_ref[...], w_ref[...], preferred_element_type=jnp.float32)
    res = res * sc_ref[...]
    o_ref[0] = res[:, :HHID]
    o_ref[1] = res[:, HHID:]


_mm_call = pl.pallas_call(
    _mm_split_body,
    grid=(P // BLK,),
    in_specs=[
        pl.BlockSpec((BLK, HID), lambda i: (i, 0)),
        pl.BlockSpec((BLK, 1), lambda i: (i, 0)),
        pl.BlockSpec((HID, HID), lambda i: (0, 0)),
    ],
    out_specs=pl.BlockSpec((2, BLK, HHID), lambda i: (0, i, 0)),
    out_shape=jax.ShapeDtypeStruct((2, P, HHID), jnp.float32),
)


def _mid_body(raw_ref, d_ref, b_ref, w_ref, p_ref, o_ref):
    lo = jnp.maximum(raw_ref[0] * d_ref[...] + b_ref[:, :HHID], 0.0)
    hi = jnp.maximum(raw_ref[1] * d_ref[...] + b_ref[:, HHID:], 0.0)
    x = jnp.concatenate([lo, hi], axis=1)
    z = jnp.dot(x, w_ref[...], preferred_element_type=jnp.float32)
    z = z * p_ref[...]
    o_ref[0] = z[:, :HHID]
    o_ref[1] = z[:, HHID:]


_mid_call = pl.pallas_call(
    _mid_body,
    grid=(P // BLK,),
    in_specs=[
        pl.BlockSpec((2, BLK, HHID), lambda i: (0, i, 0)),
        pl.BlockSpec((BLK, 1), lambda i: (i, 0)),
        pl.BlockSpec((1, HID), lambda i: (0, 0)),
        pl.BlockSpec((HID, HID), lambda i: (0, 0)),
        pl.BlockSpec((BLK, 1), lambda i: (i, 0)),
    ],
    out_specs=pl.BlockSpec((2, BLK, HHID), lambda i: (0, i, 0)),
    out_shape=jax.ShapeDtypeStruct((2, P, HHID), jnp.float32),
)


def _final_body(raw_ref, d_ref, b_ref, wl_ref, bl_ref, o_ref, mx_ref):
    i = pl.program_id(0)
    lo = jnp.maximum(raw_ref[0] * d_ref[...] + b_ref[:, :HHID], 0.0)
    hi = jnp.maximum(raw_ref[1] * d_ref[...] + b_ref[:, HHID:], 0.0)
    x = jnp.concatenate([lo, hi], axis=1)
    rowid = i * BLK + lax.broadcasted_iota(jnp.int32, (BLK, 1), 0)
    x = jnp.where(rowid < NE, x, -jnp.inf)
    bm = jnp.max(x, axis=0, keepdims=True)

    @pl.when(i == 0)
    def _():
        mx_ref[0:1] = bm

    @pl.when(i > 0)
    def _():
        mx_ref[0:1] = jnp.maximum(mx_ref[0:1], bm)

    @pl.when(i == P // BLK - 1)
    def _():
        prod = mx_ref[0:1] * wl_ref[...].reshape(1, HID)
        o_ref[...] = jnp.sum(prod, axis=1, keepdims=True) + bl_ref[...]


_final_call = pl.pallas_call(
    _final_body,
    grid=(P // BLK,),
    in_specs=[
        pl.BlockSpec((2, BLK, HHID), lambda i: (0, i, 0)),
        pl.BlockSpec((BLK, 1), lambda i: (i, 0)),
        pl.BlockSpec((1, HID), lambda i: (0, 0)),
        pl.BlockSpec((HID, 1), lambda i: (0, 0)),
        pl.BlockSpec((1, 1), lambda i: (0, 0)),
    ],
    out_specs=pl.BlockSpec((1, 1), lambda i: (0, 0)),
    out_shape=jax.ShapeDtypeStruct((1, 1), jnp.float32),
    scratch_shapes=[pltpu.VMEM((8, HID), jnp.float32)],
)


def kernel(x_0, node_idx, edge_idx, W0_l0, b01_l0, W1_l0, b10_l0,
           W0_l1, b01_l1, W1_l1, b10_l1, W_lin, b_lin):
    icat = jnp.concatenate(
        [node_idx.astype(jnp.int32), edge_idx.astype(jnp.int32)]
    )
    cards = _cards_call(icat)
    invs = _inv_call(icat, cards)
    node_card = cards[:P].reshape(P, 1)
    edge_card = cards[P:].reshape(P, 1)
    d0li = invs[:P].reshape(P, 1)
    d1li = invs[P:].reshape(P, 1)

    x0p = jnp.pad(x_0, ((0, P - NE), (0, 0)))
    b01_0 = b01_l0.reshape(1, HID)
    b10_0 = b10_l0.reshape(1, HID)
    b01_1 = b01_l1.reshape(1, HID)
    b10_1 = b10_l1.reshape(1, HID)

    xm = _mm_call(x0p, node_card, W0_l0)
    r1 = _spass_n2e(xm.reshape(2 * P, HHID), icat)
    ym = _mid_call(r1.reshape(2, P, HHID), d1li, b01_0, W1_l0, edge_card)
    r2 = _spass_e2n(ym.reshape(2 * P, HHID), icat)
    xm2 = _mid_call(r2.reshape(2, P, HHID), d0li, b10_0, W0_l1, node_card)
    r3 = _spass_n2e(xm2.reshape(2 * P, HHID), icat)
    ym2 = _mid_call(r3.reshape(2, P, HHID), d1li, b01_1, W1_l1, edge_card)
    r4 = _spass_e2n(ym2.reshape(2 * P, HHID), icat)
    out = _final_call(
        r4.reshape(2, P, HHID), d0li, b10_1, W_lin, b_lin.reshape(1, 1)
    )
    return out.reshape(1)
